# Initial kernel scaffold; baseline (speedup 1.0000x reference)
import jax
import jax.numpy as jnp
from jax.experimental import pallas as pl


def kernel(x, edge_index, batch, params):
    raise NotImplementedError("write your pallas kernel here")


# speedup vs baseline: 4.5850x; 4.5850x over previous
import jax
import jax.numpy as jnp
from jax import lax
from jax.experimental import pallas as pl
from jax.experimental.pallas import tpu as pltpu
from jax.experimental.pallas import tpu_sc as plsc

N = 50000
DIM = 32
G = 1024
OUT = 128
FP = 96

NC, NS = 2, 16
NW = NC * NS
EC = 128
KI = 10

ACC_ROWS = 50176
ZCHUNK = ACC_ROWS // NS // 14
ROWS_PER_SUB = N // NS

BLK = 2000
NB = N // BLK
BLKF = 1000
NBF = N // BLKF


def _sc_scatter(h, src_w, dst_w):
    K = src_w.shape[1]
    mesh = plsc.VectorSubcoreMesh(core_axis_name="c", subcore_axis_name="s",
                                  num_cores=NC, num_subcores=NS)

    def body(h_hbm, src_hbm, dst_hbm, out_hbm, acc_sh, src_v, dst_v, rowbuf,
             zbuf, sem):
        c = lax.axis_index("c")
        s = lax.axis_index("s")
        w = c * NS + s
        zero16 = jnp.zeros((16,), jnp.float32)

        def zrow(i, carry):
            zbuf[i, pl.ds(0, 16)] = zero16
            zbuf[i, pl.ds(16, 16)] = zero16
            return carry
        lax.fori_loop(0, ZCHUNK, zrow, 0)

        def zdma(i, carry):
            pltpu.sync_copy(zbuf,
                            acc_sh.at[pl.ds(s * 14 * ZCHUNK + i * ZCHUNK,
                                            ZCHUNK), :])
            return carry
        lax.fori_loop(0, 14, zdma, 0)

        plsc.subcore_barrier()

        def group(g, carry):
            pltpu.sync_copy(src_hbm.at[w, pl.ds(g * KI, KI)], src_v)
            pltpu.sync_copy(dst_hbm.at[w, pl.ds(g * KI, KI)], dst_v)

            def edge(j, carry2):
                pltpu.async_copy(h_hbm.at[src_v.at[j]], rowbuf, sem).wait()
                pltpu.sync_copy(rowbuf, acc_sh.at[dst_v.at[j]], add=True)
                return carry2
            lax.fori_loop(0, KI, edge, 0)
            return carry
        lax.fori_loop(0, K // KI, group, 0)
        plsc.subcore_barrier()

        pltpu.sync_copy(acc_sh.at[pl.ds(s * ROWS_PER_SUB, ROWS_PER_SUB), :],
                        out_hbm.at[c, s])

    f = pl.kernel(
        body,
        out_type=jax.ShapeDtypeStruct((NC, NS, ROWS_PER_SUB, DIM),
                                      jnp.float32),
        mesh=mesh,
        scratch_types=[
            pltpu.VMEM_SHARED((ACC_ROWS, DIM), jnp.float32),
            pltpu.VMEM((KI, EC), jnp.int32),
            pltpu.VMEM((KI, EC), jnp.int32),
            pltpu.VMEM((EC, DIM), jnp.float32),
            pltpu.VMEM((ZCHUNK, DIM), jnp.float32),
            pltpu.SemaphoreType.DMA,
        ],
        compiler_params=pltpu.CompilerParams(use_tc_tiling_on_sc=False),
    )
    return f(h, src_w, dst_w)


def _mlp(h_blk, agg_blk, w1_ref, b1_ref, w2_ref, b2_ref):
    z = h_blk + agg_blk
    z1 = jnp.maximum(jnp.dot(z, w1_ref[...],
                             preferred_element_type=jnp.float32)
                     + b1_ref[...], 0.0)
    r = jnp.maximum(jnp.dot(z1, w2_ref[...],
                            preferred_element_type=jnp.float32)
                    + b2_ref[...], 0.0)
    return r


def _tc_layer(h, accs, w1, b1, w2, b2, gamma, beta):
    f_in = h.shape[1]
    nch = len(accs)

    def body(*refs):
        h_ref = refs[0]
        acc_refs = refs[1:1 + nch]
        b1_ref, w2_ref, b2_ref, g_ref, be_ref, w1_ref = refs[1 + nch:7 + nch]
        o_ref, r_scr, st_scr = refs[7 + nch:]
        p = pl.program_id(0)
        j = pl.program_id(1)

        @pl.when(p == 0)
        def _phase0():
            agg = jnp.concatenate(
                [a[0] + a[1] for a in acc_refs], axis=1)
            r = _mlp(h_ref[...], agg, w1_ref, b1_ref, w2_ref, b2_ref)
            r_scr[pl.ds(j * BLK, BLK), :] = r

            @pl.when(j == 0)
            def _init():
                st_scr[...] = jnp.zeros_like(st_scr)
            st_scr[0:1, :] += jnp.sum(r, axis=0, keepdims=True)
            st_scr[1:2, :] += jnp.sum(r * r, axis=0, keepdims=True)

        @pl.when(p == 1)
        def _phase1():
            mu = st_scr[0:1, :] * (1.0 / N)
            var = st_scr[1:2, :] * (1.0 / N) - mu * mu
            a = g_ref[...] / jnp.sqrt(var + 1e-5)
            shift = be_ref[...] - mu * a
            o_ref[...] = r_scr[pl.ds(j * BLK, BLK), :] * a + shift

    full = lambda shape: pl.BlockSpec(shape, lambda p, j: tuple(0 for _ in shape))
    return pl.pallas_call(
        body,
        grid=(2, NB),
        in_specs=[pl.BlockSpec((BLK, f_in),
                               lambda p, j: (jnp.where(p == 0, j, NB - 1), 0))]
        + [pl.BlockSpec((NC, BLK, DIM),
                        lambda p, j: (0, jnp.where(p == 0, j, NB - 1), 0))
           for _ in range(nch)]
        + [full((1, DIM)), full((DIM, DIM)), full((1, DIM)),
           full((1, DIM)), full((1, DIM)), full((f_in, DIM))],
        out_specs=pl.BlockSpec((BLK, DIM),
                               lambda p, j: (jnp.where(p == 1, j, 0), 0)),
        out_shape=jax.ShapeDtypeStruct((N, DIM), jnp.float32),
        scratch_shapes=[pltpu.VMEM((N, DIM), jnp.float32),
                        pltpu.VMEM((2, DIM), jnp.float32)],
        compiler_params=pltpu.CompilerParams(
            dimension_semantics=("arbitrary", "arbitrary")),
    )(h, *accs, b1, w2, b2, gamma, beta, w1)


def _tc_final(h, acc, w1, b1, w2, b2, gamma, beta, batch3, wfc, bfc):
    def body(h_ref, acc_ref, b1_ref, w2_ref, b2_ref, g_ref, be_ref, bt_ref,
             wfc_ref, bfc_ref, w1_ref, o_ref, pool_scr, cnt_scr, st_scr):
        j = pl.program_id(0)
        r = _mlp(h_ref[...], acc_ref[0] + acc_ref[1],
                 w1_ref, b1_ref, w2_ref, b2_ref)

        @pl.when(j == 0)
        def _init():
            st_scr[...] = jnp.zeros_like(st_scr)
            pool_scr[...] = jnp.zeros_like(pool_scr)
            cnt_scr[...] = jnp.zeros_like(cnt_scr)

        st_scr[0:1, :] += jnp.sum(r, axis=0, keepdims=True)
        st_scr[1:2, :] += jnp.sum(r * r, axis=0, keepdims=True)

        ids = bt_ref[0, 0, :]
        seg = lax.broadcasted_iota(jnp.int32, (G, BLKF), 0)
        oh = (seg == ids[None, :]).astype(jnp.float32)
        pool_scr[...] += jnp.dot(oh, r, preferred_element_type=jnp.float32)
        cnt_scr[...] += jnp.sum(oh, axis=1, keepdims=True)

        @pl.when(j == NBF - 1)
        def _finish():
            mu = st_scr[0:1, :] * (1.0 / N)
            var = st_scr[1:2, :] * (1.0 / N) - mu * mu
            a = g_ref[...] / jnp.sqrt(var + 1e-5)
            shift = be_ref[...] - mu * a
            pooled = pool_scr[...] * a + cnt_scr[...] * shift
            o_ref[...] = jnp.maximum(
                jnp.dot(pooled, wfc_ref[...],
                        preferred_element_type=jnp.float32) + bfc_ref[...],
                0.0)

    full = lambda shape: pl.BlockSpec(shape, lambda j: tuple(0 for _ in shape))
    return pl.pallas_call(
        body,
        grid=(NBF,),
        in_specs=[
            pl.BlockSpec((BLKF, DIM), lambda j: (j, 0)),
            pl.BlockSpec((NC, BLKF, DIM), lambda j: (0, j, 0)),
            full((1, DIM)), full((DIM, DIM)), full((1, DIM)),
            full((1, DIM)), full((1, DIM)),
            pl.BlockSpec((1, 1, BLKF), lambda j: (j, 0, 0)),
            full((DIM, OUT)), full((1, OUT)), full((DIM, DIM)),
        ],
        out_specs=full((G, OUT)),
        out_shape=jax.ShapeDtypeStruct((G, OUT), jnp.float32),
        scratch_shapes=[pltpu.VMEM((G, DIM), jnp.float32),
                        pltpu.VMEM((G, 1), jnp.float32),
                        pltpu.VMEM((2, DIM), jnp.float32)],
        compiler_params=pltpu.CompilerParams(
            dimension_semantics=("arbitrary",)),
    )(h, acc, b1, w2, b2, gamma, beta, batch3, wfc, bfc, w1)


def _row(v):
    return v.reshape(1, -1)


def kernel(x, edge_index, batch, params):
    e = edge_index.shape[1]
    per = NW * EC * KI
    k = (-(-e // per)) * KI
    ep = k * NW * EC
    src = jnp.concatenate(
        [edge_index[0], jnp.zeros((ep - e,), jnp.int32)]).reshape(NW, k, EC)
    dst = jnp.concatenate(
        [edge_index[1], jnp.full((ep - e,), N, jnp.int32)]).reshape(NW, k, EC)
    batch3 = batch.reshape(NBF, 1, BLKF)

    f_in = x.shape[1]
    xp = jnp.pad(x, ((0, 0), (0, FP - f_in)))
    w1p = jnp.pad(params['conv1']['W1'], ((0, FP - f_in), (0, 0)))

    accs = [_sc_scatter(xp[:, t * DIM:(t + 1) * DIM], src, dst)
            .reshape(NC, N, DIM) for t in range(FP // DIM)]
    cp, bp = params['conv1'], params['bn1']
    h = _tc_layer(xp, accs, w1p, _row(cp['b1']), cp['W2'], _row(cp['b2']),
                  _row(bp['gamma']), _row(bp['beta']))
    for i in range(2, 5):
        acc = _sc_scatter(h, src, dst).reshape(NC, N, DIM)
        cp, bp = params[f'conv{i}'], params[f'bn{i}']
        h = _tc_layer(h, [acc], cp['W1'], _row(cp['b1']), cp['W2'],
                      _row(cp['b2']), _row(bp['gamma']), _row(bp['beta']))
    acc = _sc_scatter(h, src, dst).reshape(NC, N, DIM)
    cp, bp = params['conv5'], params['bn5']
    return _tc_final(h, acc, cp['W1'], _row(cp['b1']), cp['W2'],
                     _row(cp['b2']), _row(bp['gamma']), _row(bp['beta']),
                     batch3, params['fc']['W'], _row(params['fc']['b']))
